# baseline (speedup 1.0000x reference)
import functools

import jax
import jax.numpy as jnp
from jax import lax
from jax.experimental import pallas as pl
from jax.experimental.pallas import tpu as pltpu
from jax.experimental.pallas import tpu_sc as plsc

NC = 2
NS = 16
NW = NC * NS
LANES = 16


def _worker_id():
    return lax.axis_index("s") * NC + lax.axis_index("c")


def _splat(val):
    return jnp.full((LANES,), val, dtype=jnp.int32)


def _make_reformat(V, D):
    R2 = V // 2
    NT = V // 128
    TROWS = R2 - NT * 64
    RPT = 64
    ITERS = (NT + NW - 1) // NW
    KV = D // LANES

    mesh = plsc.VectorSubcoreMesh(core_axis_name="c", subcore_axis_name="s")

    @functools.partial(
        pl.kernel,
        mesh=mesh,
        out_type=jax.ShapeDtypeStruct((R2, 2 * D), jnp.float32),
        compiler_params=pltpu.CompilerParams(
            use_tc_tiling_on_sc=True, needs_layout_passes=False
        ),
        scratch_types=[
            pltpu.VMEM((2, D, 128), jnp.float32),
            pltpu.VMEM((2, RPT, 2 * D), jnp.float32),
            pltpu.VMEM((TROWS, 2 * D), jnp.float32),
            pltpu.SemaphoreType.DMA,
            pltpu.SemaphoreType.DMA,
        ],
    )
    def reformat(embT_hbm, tail2_hbm, emb2_hbm, in_v, out_v, tail_v,
                 isem, wsem):
        wid = _worker_id()
        iota = lax.iota(jnp.int32, LANES)

        def in_copy(vt, buf):
            return pltpu.make_async_copy(
                embT_hbm.at[pl.ds(0, D), pl.ds(vt * 128, 128)],
                in_v.at[buf],
                isem,
            )

        def out_copy(vt, buf):
            return pltpu.make_async_copy(
                out_v.at[buf],
                emb2_hbm.at[pl.ds(vt * RPT, RPT)],
                wsem,
            )

        @pl.when(wid == NW - 1)
        def _():
            pltpu.sync_copy(tail2_hbm, tail_v)
            pltpu.sync_copy(tail_v, emb2_hbm.at[pl.ds(NT * RPT, TROWS)])

        in_copy(wid, 0).start()

        def body(it, carry):
            vt = wid + it * NW
            b0 = lax.rem(it, 2)
            b1 = lax.rem(it + 1, 2)
            nvt = vt + NW

            @pl.when(it > 1)
            def _():
                out_copy(vt - 2 * NW, b0).wait()

            @pl.when(nvt < NT)
            def _():
                in_copy(nvt, b1).start()

            @pl.when(vt < NT)
            def _():
                in_copy(vt, b0).wait()

                @plsc.parallel_loop(0, RPT, unroll=2)
                def row_body(i):
                    for p in range(2):
                        col = _splat(2 * i + p)
                        for k in range(KV):
                            vec = plsc.load_gather(
                                in_v,
                                [_splat(b0), iota + (k * LANES), col],
                            )
                            out_v[b0, i, pl.ds(p * D + k * LANES, LANES)] = vec

                out_copy(vt, b0).start()
            return carry

        lax.fori_loop(0, ITERS, body, 0)
        lv2 = wid + (ITERS - 2) * NW
        @pl.when(lv2 < NT)
        def _():
            out_copy(lv2, (ITERS - 2) % 2).wait()
        lv = wid + (ITERS - 1) * NW
        @pl.when(lv < NT)
        def _():
            out_copy(lv, (ITERS - 1) % 2).wait()

    return reformat


def _make_lookup(B, T, V, D):
    NPOS = 128 // D
    NPR = (T // NPOS + 7) // 8 * 8
    KV = D // LANES

    mesh = plsc.VectorSubcoreMesh(core_axis_name="c", subcore_axis_name="s")

    @functools.partial(
        pl.kernel,
        mesh=mesh,
        out_type=jax.ShapeDtypeStruct((T, D, B), jnp.float32),
        compiler_params=pltpu.CompilerParams(
            use_tc_tiling_on_sc=True, needs_layout_passes=False
        ),
        scratch_types=[
            pltpu.VMEM((2, 128), jnp.int32),
            pltpu.VMEM((2, 128, 2 * D), jnp.float32),
            pltpu.VMEM((2, D, 128), jnp.float32),
            pltpu.VMEM((NPR, 2 * D), jnp.float32),
            pltpu.SemaphoreType.DMA,
            pltpu.SemaphoreType.DMA,
            pltpu.SemaphoreType.DMA,
        ],
    )
    def lookup(xT_hbm, emb2_hbm, pos2_hbm, outT_hbm, idx_v, slab_v, blk_v,
               pos_v, isem, gsem, wsem):
        wid = _worker_id()
        bcol = wid * 128
        iota = lax.iota(jnp.int32, LANES)
        pltpu.sync_copy(pos2_hbm.at[pl.ds(0, NPR)], pos_v)

        def idx_copy(t, buf):
            return pltpu.make_async_copy(
                xT_hbm.at[t, pl.ds(bcol, 128)], idx_v.at[buf], isem
            )

        def gather_copy(buf):
            return pltpu.make_async_copy(
                emb2_hbm.at[idx_v.at[buf]], slab_v.at[buf], gsem
            )

        def blk_copy(t, buf):
            return pltpu.make_async_copy(
                blk_v.at[buf],
                outT_hbm.at[t, pl.ds(0, D), pl.ds(bcol, 128)],
                wsem,
            )

        def read_par(buf):
            return tuple(
                idx_v[buf, pl.ds(g * LANES, LANES)] & 1 for g in range(8)
            )

        def shift_ids(buf):
            for g in range(8):
                sl = pl.ds(g * LANES, LANES)
                idx_v[buf, sl] = lax.shift_right_logical(idx_v[buf, sl], 1)

        idx_copy(0, 0).start()
        idx_copy(0, 0).wait()
        par0 = read_par(0)
        shift_ids(0)
        gather_copy(0).start()

        def step(t, par):
            bf0 = lax.rem(t, 2)
            bf1 = lax.rem(t + 1, 2)

            @pl.when(t + 1 < T)
            def _():
                idx_copy(t + 1, bf1).start()

            gather_copy(bf0).wait()

            @pl.when(t + 1 < T)
            def _():
                idx_copy(t + 1, bf1).wait()

            nxt = read_par(bf1)

            @pl.when(t + 1 < T)
            def _():
                shift_ids(bf1)
                gather_copy(bf1).start()

            @pl.when(t > 1)
            def _():
                blk_copy(t - 2, bf0).wait()

            prow = _splat(t // NPOS)
            pcol = lax.rem(t, NPOS) * D
            colbase = [par[g] * D for g in range(8)]
            rowbase = [iota + (g * LANES) for g in range(8)]
            for d in range(D):
                pvec = plsc.load_gather(pos_v, [prow, _splat(pcol + d)])
                for g in range(8):
                    vec = plsc.load_gather(
                        slab_v, [_splat(bf0), rowbase[g], colbase[g] + d]
                    )
                    blk_v[bf0, d, pl.ds(g * LANES, LANES)] = vec + pvec

            blk_copy(t, bf0).start()
            return nxt

        lax.fori_loop(0, T, step, par0)
        blk_copy(T - 2, (T - 2) % 2).wait()
        blk_copy(T - 1, (T - 1) % 2).wait()

    return lookup


def kernel(x, emb, posenc):
    B, T = x.shape
    V, D = emb.shape
    embT = emb.T
    xT = x.astype(jnp.int32).T
    pos2 = posenc.reshape(posenc.shape[0] * D // 128, 128)
    NTF = V // 128
    tail2 = emb[NTF * 128:].reshape(-1, 128)
    emb2 = _make_reformat(V, D)(embT, tail2)
    outT = _make_lookup(B, T, V, D)(xT, emb2, pos2)
    return jnp.transpose(outT, (2, 0, 1))
